# baseline (speedup 1.0000x reference)
import jax
import jax.numpy as jnp
from jax import lax
from jax.experimental import pallas as pl
from jax.experimental.pallas import tpu as pltpu
from jax.experimental.pallas import tpu_sc as plsc

NC = 2
NS = 16
NW = NC * NS
L = 16

FMIN = -3.4e38
FMAX = 3.4e38


def _node_pre_body(x_ref, win_ref, bin_ref, w1_ref, w2_ref, h_ref, at_ref, bt_ref):
    h = jnp.maximum(jnp.dot(x_ref[...], win_ref[...]) + bin_ref[...], 0.0)
    h_ref[...] = h
    at_ref[...] = lax.dot_general(w1_ref[...], h, (((0,), (1,)), ((), ())))
    bt_ref[...] = lax.dot_general(w2_ref[...], h, (((0,), (1,)), ((), ())))


def _edge_c_body(eat_ref, w3_ref, bmsg_ref, ct_ref):
    ct_ref[...] = (
        lax.dot_general(w3_ref[...], eat_ref[...], (((0,), (0,)), ((), ())))
        + bmsg_ref[...]
    )


def _finalize_body(s0_ref, s1_ref, q0_ref, q1_ref, x0_ref, x1_ref, n0_ref, n1_ref,
                   degp_ref, h_ref,
                   wu0_ref, wu1_ref, wu2_ref, wu3_ref, wu4_ref, wu5_ref, wu6_ref,
                   wu7_ref, bupd_ref, wp_ref, bp_ref, hp_ref):
    deg = jnp.sum(degp_ref[...], axis=0, keepdims=True)
    rden = 1.0 / jnp.maximum(deg, 1.0)
    nonempty = deg > 0.0

    def stats(s_ref, q_ref, x_ref, n_ref):
        mean = s_ref[...] * rden
        var = jnp.maximum(q_ref[...] * rden - mean * mean, 0.0)
        std = jnp.sqrt(var + 1e-5)
        mx = jnp.where(nonempty, x_ref[...], 0.0)
        mn = jnp.where(nonempty, n_ref[...], 0.0)
        return mean, mx, mn, std

    mean0, mx0, mn0, std0 = stats(s0_ref, q0_ref, x0_ref, n0_ref)
    mean1, mx1, mn1, std1 = stats(s1_ref, q1_ref, x1_ref, n1_ref)

    dn = (((0,), (0,)), ((), ()))
    upd = (lax.dot_general(mean0, wu0_ref[...], dn)
           + lax.dot_general(mean1, wu1_ref[...], dn)
           + lax.dot_general(mx0, wu2_ref[...], dn)
           + lax.dot_general(mx1, wu3_ref[...], dn)
           + lax.dot_general(mn0, wu4_ref[...], dn)
           + lax.dot_general(mn1, wu5_ref[...], dn)
           + lax.dot_general(std0, wu6_ref[...], dn)
           + lax.dot_general(std1, wu7_ref[...], dn))
    hout = h_ref[...] + jnp.maximum(upd + bupd_ref[...], 0.0)
    hp_ref[...] = (
        lax.dot_general(wp_ref[...], hout, (((0,), (1,)), ((), ()))) + bp_ref[...]
    )


def _sqrt_body(d2_ref, out_ref):
    out_ref[...] = jnp.sqrt(d2_ref[...] + 1e-12)


def _rmw_extreme(acc_v, idxv, val, is_max):
    def cond(carry):
        return carry[0] > 0

    def body(carry):
        _, todo = carry
        cur = plsc.load_gather(acc_v, [idxv])
        new = jnp.maximum(cur, val) if is_max else jnp.minimum(cur, val)
        plsc.store_scatter(acc_v, [idxv], new, mask=todo)
        chk = plsc.load_gather(acc_v, [idxv])
        ok = (chk >= val) if is_max else (chk <= val)
        todo2 = jnp.logical_and(todo, jnp.logical_not(ok))
        return jnp.sum(todo2.astype(jnp.int32)), todo2

    lax.while_loop(cond, body, (jnp.int32(L), jnp.ones((L,), jnp.bool_)))


def _make_deg_body(n_nodes, n_edges):
    epw = n_edges // NW

    def body(dst_hbm, degp_hbm, idx_v, acc_v):
        wid = lax.axis_index("c") * NS + lax.axis_index("s")
        zz = jnp.zeros((L,), jnp.float32)

        def zloop(i, c):
            acc_v[pl.ds(i * L, L)] = zz
            return c
        lax.fori_loop(0, n_nodes // L, zloop, 0)

        pltpu.sync_copy(dst_hbm.at[pl.ds(wid * epw, epw)], idx_v)
        ones = jnp.ones((L,), jnp.float32)

        def eloop(i, c):
            plsc.addupdate_scatter(acc_v, [idx_v[pl.ds(i * L, L)]], ones)
            return c
        lax.fori_loop(0, epw // L, eloop, 0)
        pltpu.sync_copy(acc_v, degp_hbm.at[pl.ds(wid * n_nodes, n_nodes)])

    return body


def _make_edge_pass_body(pass_off, n_nodes, n_edges, chunk):
    def body(at_hbm, bt_hbm, ct_hbm, ei_hbm,
             sum_hbm, sq_hbm, mx_hbm, mn_hbm,
             a0_v, a1_v, b0_v, b1_v,
             s0_v, s1_v, q0_v, q1_v, x0_v, x1_v, n0_v, n1_v,
             ei_b, c_b, sem0, sem1):
        wid = lax.axis_index("c") * NS + lax.axis_index("s")
        f_loc = 2 * wid
        f_glob = pass_off + f_loc

        zz = jnp.zeros((L,), jnp.float32)
        lo = jnp.full((L,), FMIN, jnp.float32)
        hi = jnp.full((L,), FMAX, jnp.float32)

        def zloop(i, c):
            sl = pl.ds(i * L, L)
            s0_v[sl] = zz
            s1_v[sl] = zz
            q0_v[sl] = zz
            q1_v[sl] = zz
            x0_v[sl] = lo
            x1_v[sl] = lo
            n0_v[sl] = hi
            n1_v[sl] = hi
            return c
        lax.fori_loop(0, n_nodes // L, zloop, 0)

        nn = n_nodes
        pltpu.sync_copy(at_hbm.at[pl.ds(f_glob * nn, nn)], a0_v)
        pltpu.sync_copy(at_hbm.at[pl.ds((f_glob + 1) * nn, nn)], a1_v)
        pltpu.sync_copy(bt_hbm.at[pl.ds(f_glob * nn, nn)], b0_v)
        pltpu.sync_copy(bt_hbm.at[pl.ds((f_glob + 1) * nn, nn)], b1_v)

        fpair = pass_off // 2 + wid
        c_base = fpair * 2 * n_edges

        def issue(ci, par):
            base = ci * chunk
            half = pl.ds(par * 2 * chunk, 2 * chunk)
            pltpu.async_copy(ei_hbm.at[pl.ds(2 * base, 2 * chunk)], ei_b.at[half], sem0)
            pltpu.async_copy(
                ct_hbm.at[pl.ds(c_base + 2 * base, 2 * chunk)], c_b.at[half], sem1)

        issue(0, 0)
        nchunks = n_edges // chunk

        def chunk_loop(ci, c):
            par = lax.rem(ci, 2)
            base = ci * chunk
            half = pl.ds(par * 2 * chunk, 2 * chunk)

            @pl.when(ci + 1 < nchunks)
            def _prefetch():
                issue(ci + 1, 1 - par)

            pltpu.make_async_copy(
                ei_hbm.at[pl.ds(2 * base, 2 * chunk)], ei_b.at[half], sem0).wait()
            pltpu.make_async_copy(
                ct_hbm.at[pl.ds(c_base + 2 * base, 2 * chunk)], c_b.at[half], sem1).wait()

            ii2 = 2 * jnp.arange(L, dtype=jnp.int32)

            def prep(off):
                j = par * 2 * chunk + 2 * off
                srcv = plsc.load_gather(ei_b, [j + ii2])
                dstv = plsc.load_gather(ei_b, [j + 1 + ii2])
                cnt, _ = plsc.scan_count(dstv)
                neq = plsc.all_reduce_population_count(cnt == cnt[0])
                dup = neq[0] < L
                m0 = jnp.maximum(
                    plsc.load_gather(a0_v, [srcv])
                    + plsc.load_gather(b0_v, [dstv])
                    + plsc.load_gather(c_b, [j + ii2]), 0.0)
                m1 = jnp.maximum(
                    plsc.load_gather(a1_v, [srcv])
                    + plsc.load_gather(b1_v, [dstv])
                    + plsc.load_gather(c_b, [j + 1 + ii2]), 0.0)
                plsc.addupdate_scatter(s0_v, [dstv], m0)
                plsc.addupdate_scatter(q0_v, [dstv], m0 * m0)
                plsc.addupdate_scatter(s1_v, [dstv], m1)
                plsc.addupdate_scatter(q1_v, [dstv], m1 * m1)
                return dstv, m0, m1, dup

            def rmw(dstv, m0, m1, dup):
                def _fast():
                    cx0 = plsc.load_gather(x0_v, [dstv])
                    cn0 = plsc.load_gather(n0_v, [dstv])
                    cx1 = plsc.load_gather(x1_v, [dstv])
                    cn1 = plsc.load_gather(n1_v, [dstv])
                    plsc.store_scatter(x0_v, [dstv], jnp.maximum(cx0, m0))
                    plsc.store_scatter(n0_v, [dstv], jnp.minimum(cn0, m0))
                    plsc.store_scatter(x1_v, [dstv], jnp.maximum(cx1, m1))
                    plsc.store_scatter(n1_v, [dstv], jnp.minimum(cn1, m1))

                _fast()

                @pl.when(dup)
                def _slow():
                    _rmw_extreme(x0_v, dstv, m0, True)
                    _rmw_extreme(n0_v, dstv, m0, False)
                    _rmw_extreme(x1_v, dstv, m1, True)
                    _rmw_extreme(n1_v, dstv, m1, False)

            def vec_loop(i, cc):
                ts = [prep(i * 4 * L + k * L) for k in range(4)]
                for t in ts:
                    rmw(*t)
                return cc
            lax.fori_loop(0, chunk // (4 * L), vec_loop, 0)
            return c
        lax.fori_loop(0, nchunks, chunk_loop, 0)

        pltpu.sync_copy(s0_v, sum_hbm.at[pl.ds(f_loc * nn, nn)])
        pltpu.sync_copy(s1_v, sum_hbm.at[pl.ds((f_loc + 1) * nn, nn)])
        pltpu.sync_copy(q0_v, sq_hbm.at[pl.ds(f_loc * nn, nn)])
        pltpu.sync_copy(q1_v, sq_hbm.at[pl.ds((f_loc + 1) * nn, nn)])
        pltpu.sync_copy(x0_v, mx_hbm.at[pl.ds(f_loc * nn, nn)])
        pltpu.sync_copy(x1_v, mx_hbm.at[pl.ds((f_loc + 1) * nn, nn)])
        pltpu.sync_copy(n0_v, mn_hbm.at[pl.ds(f_loc * nn, nn)])
        pltpu.sync_copy(n1_v, mn_hbm.at[pl.ds((f_loc + 1) * nn, nn)])

    return body


def _make_pair_body(n_nodes, n_pairs):
    ppw = n_pairs // NW

    def body(hp_hbm, pi_hbm, d2_hbm, hx_v, hy_v, hz_v, i0_v, i1_v, out_v):
        wid = lax.axis_index("c") * NS + lax.axis_index("s")
        base = wid * ppw
        nn = n_nodes
        pltpu.sync_copy(hp_hbm.at[pl.ds(0, nn)], hx_v)
        pltpu.sync_copy(hp_hbm.at[pl.ds(nn, nn)], hy_v)
        pltpu.sync_copy(hp_hbm.at[pl.ds(2 * nn, nn)], hz_v)
        pltpu.sync_copy(pi_hbm.at[pl.ds(base, ppw)], i0_v)
        pltpu.sync_copy(pi_hbm.at[pl.ds(n_pairs + base, ppw)], i1_v)

        def loop(i, c):
            sl = pl.ds(i * L, L)
            i0 = i0_v[sl]
            i1 = i1_v[sl]
            dx = plsc.load_gather(hx_v, [i0]) - plsc.load_gather(hx_v, [i1])
            dy = plsc.load_gather(hy_v, [i0]) - plsc.load_gather(hy_v, [i1])
            dz = plsc.load_gather(hz_v, [i0]) - plsc.load_gather(hz_v, [i1])
            out_v[sl] = dx * dx + dy * dy + dz * dz
            return c
        lax.fori_loop(0, ppw // L, loop, 0)
        pltpu.sync_copy(out_v, d2_hbm.at[pl.ds(base, ppw)])

    return body


_sc_mesh_cache = []


def _sc_mesh():
    if not _sc_mesh_cache:
        _sc_mesh_cache.append(plsc.VectorSubcoreMesh(
            core_axis_name="c", subcore_axis_name="s",
            num_cores=NC, num_subcores=NS,
        ))
    return _sc_mesh_cache[0]


def _full(shape):
    return pl.BlockSpec(shape, lambda i: (0, 0))


def kernel(x, pos, edge_index, edge_attr, pairwise_indices,
           W_in, b_in, W_msg, b_msg, W_upd, b_upd, W_proj, b_proj):
    n, d = x.shape
    e, ed = edge_attr.shape
    h_dim = W_in.shape[1]
    p = pairwise_indices.shape[1]
    del pos

    nb = 1024
    eb = 6400
    chunk = 640
    n_pad = ((n + nb - 1) // nb) * nb
    x = jnp.pad(x, ((0, n_pad - n), (0, 0)))

    f32 = jnp.float32
    w1 = W_msg[:h_dim]
    w2 = W_msg[h_dim:2 * h_dim]
    w3 = W_msg[2 * h_dim:]
    ea_t = edge_attr.T
    bin2 = b_in.reshape(1, h_dim)
    bmsg2 = b_msg.reshape(h_dim, 1)
    bupd2 = b_upd.reshape(1, h_dim)
    wp8 = jnp.pad(W_proj, ((0, 0), (0, 8 - W_proj.shape[1])))
    bp8 = jnp.pad(b_proj, (0, 8 - b_proj.shape[0])).reshape(8, 1)

    h, a_t, b_t = pl.pallas_call(
        _node_pre_body,
        grid=(n_pad // nb,),
        in_specs=[
            pl.BlockSpec((nb, d), lambda i: (i, 0)),
            _full((d, h_dim)), _full((1, h_dim)),
            _full((h_dim, h_dim)), _full((h_dim, h_dim)),
        ],
        out_specs=[
            pl.BlockSpec((nb, h_dim), lambda i: (i, 0)),
            pl.BlockSpec((h_dim, nb), lambda i: (0, i)),
            pl.BlockSpec((h_dim, nb), lambda i: (0, i)),
        ],
        out_shape=[
            jax.ShapeDtypeStruct((n_pad, h_dim), f32),
            jax.ShapeDtypeStruct((h_dim, n_pad), f32),
            jax.ShapeDtypeStruct((h_dim, n_pad), f32),
        ],
    )(x, W_in, bin2, w1, w2)

    c_t = pl.pallas_call(
        _edge_c_body,
        grid=(e // eb,),
        in_specs=[
            pl.BlockSpec((ed, eb), lambda i: (0, i)),
            _full((ed, h_dim)), _full((h_dim, 1)),
        ],
        out_specs=pl.BlockSpec((h_dim, eb), lambda i: (0, i)),
        out_shape=jax.ShapeDtypeStruct((h_dim, e), f32),
    )(ea_t, w3, bmsg2)

    src = edge_index[0]
    dst = edge_index[1]
    ei_pair = edge_index.T.reshape(-1)
    c_pair = c_t.reshape(h_dim // 2, 2, e).transpose(0, 2, 1).reshape(-1)

    degp = pl.kernel(
        _make_deg_body(n_pad, e),
        compiler_params=pltpu.CompilerParams(needs_layout_passes=False),
        out_type=jax.ShapeDtypeStruct((NW * n_pad,), f32),
        mesh=_sc_mesh(),
        scratch_types=[
            pltpu.VMEM((e // NW,), jnp.int32),
            pltpu.VMEM((n_pad,), f32),
        ],
    )(dst)
    degp = degp.reshape(NW, n_pad)

    def run_pass(pass_off):
        return pl.kernel(
            _make_edge_pass_body(pass_off, n_pad, e, chunk),
            compiler_params=pltpu.CompilerParams(needs_layout_passes=False),
            out_type=[jax.ShapeDtypeStruct((h_dim // 2 * n_pad,), f32)] * 4,
            mesh=_sc_mesh(),
            scratch_types=(
                [pltpu.VMEM((n_pad,), f32)] * 4
                + [pltpu.VMEM((n_pad,), f32)] * 8
                + [pltpu.VMEM((4 * chunk,), jnp.int32)]
                + [pltpu.VMEM((4 * chunk,), f32)]
                + [pltpu.SemaphoreType.DMA] * 2
            ),
        )(a_t.reshape(-1), b_t.reshape(-1), c_pair, ei_pair)

    hw = h_dim // 2
    s0, q0, x0, n0 = (r.reshape(hw, n_pad) for r in run_pass(0))
    s1, q1, x1, n1 = (r.reshape(hw, n_pad) for r in run_pass(hw))

    wu = [W_upd[i * hw:(i + 1) * hw] for i in range(8)]
    half = pl.BlockSpec((hw, nb), lambda i: (0, i))
    hp = pl.pallas_call(
        _finalize_body,
        grid=(n_pad // nb,),
        in_specs=(
            [half] * 8
            + [pl.BlockSpec((NW, nb), lambda i: (0, i)),
               pl.BlockSpec((nb, h_dim), lambda i: (i, 0))]
            + [_full((hw, h_dim))] * 8
            + [_full((1, h_dim)), _full((h_dim, 8)), _full((8, 1))]
        ),
        out_specs=pl.BlockSpec((8, nb), lambda i: (0, i)),
        out_shape=jax.ShapeDtypeStruct((8, n_pad), f32),
    )(s0, s1, q0, q1, x0, x1, n0, n1, degp, h, *wu, bupd2, wp8, bp8)

    d2 = pl.kernel(
        _make_pair_body(n_pad, p),
        compiler_params=pltpu.CompilerParams(needs_layout_passes=False),
        out_type=jax.ShapeDtypeStruct((p,), f32),
        mesh=_sc_mesh(),
        scratch_types=(
            [pltpu.VMEM((n_pad,), f32)] * 3
            + [pltpu.VMEM((p // NW,), jnp.int32)] * 2
            + [pltpu.VMEM((p // NW,), f32)]
        ),
    )(hp.reshape(-1), pairwise_indices.reshape(-1))

    rows = 2500
    dist = pl.pallas_call(
        _sqrt_body,
        out_shape=jax.ShapeDtypeStruct((rows, p // rows), f32),
    )(d2.reshape(rows, p // rows))

    return dist.reshape(p, 1)


# speedup vs baseline: 17.0410x; 17.0410x over previous
import jax
import jax.numpy as jnp
from jax import lax
from jax.experimental import pallas as pl
from jax.experimental.pallas import tpu as pltpu
from jax.experimental.pallas import tpu_sc as plsc

NC = 2
NS = 16
NW = NC * NS
L = 16

FMIN = -3.4e38
FMAX = 3.4e38


def _node_pre_body(x_ref, win_ref, bin_ref, w1_ref, w2_ref, h_ref, at_ref, bt_ref):
    h = jnp.maximum(jnp.dot(x_ref[...], win_ref[...]) + bin_ref[...], 0.0)
    h_ref[...] = h
    at_ref[...] = lax.dot_general(w1_ref[...], h, (((0,), (1,)), ((), ())))
    bt_ref[...] = lax.dot_general(w2_ref[...], h, (((0,), (1,)), ((), ())))


def _edge_c_body(eat_ref, w3_ref, bmsg_ref, ct_ref):
    ct_ref[...] = (
        lax.dot_general(w3_ref[...], eat_ref[...], (((0,), (0,)), ((), ())))
        + bmsg_ref[...]
    )


def _finalize_body(s0_ref, s1_ref, q0_ref, q1_ref, x0_ref, x1_ref, n0_ref, n1_ref,
                   degp_ref, h_ref,
                   wu0_ref, wu1_ref, wu2_ref, wu3_ref, wu4_ref, wu5_ref, wu6_ref,
                   wu7_ref, bupd_ref, wp_ref, bp_ref, hp_ref):
    deg = jnp.sum(degp_ref[...], axis=0, keepdims=True)
    rden = 1.0 / jnp.maximum(deg, 1.0)
    nonempty = deg > 0.0

    def stats(s_ref, q_ref, x_ref, n_ref):
        mean = s_ref[...] * rden
        var = jnp.maximum(q_ref[...] * rden - mean * mean, 0.0)
        std = jnp.sqrt(var + 1e-5)
        mx = jnp.where(nonempty, x_ref[...], 0.0)
        mn = jnp.where(nonempty, n_ref[...], 0.0)
        return mean, mx, mn, std

    mean0, mx0, mn0, std0 = stats(s0_ref, q0_ref, x0_ref, n0_ref)
    mean1, mx1, mn1, std1 = stats(s1_ref, q1_ref, x1_ref, n1_ref)

    dn = (((0,), (0,)), ((), ()))
    upd = (lax.dot_general(mean0, wu0_ref[...], dn)
           + lax.dot_general(mean1, wu1_ref[...], dn)
           + lax.dot_general(mx0, wu2_ref[...], dn)
           + lax.dot_general(mx1, wu3_ref[...], dn)
           + lax.dot_general(mn0, wu4_ref[...], dn)
           + lax.dot_general(mn1, wu5_ref[...], dn)
           + lax.dot_general(std0, wu6_ref[...], dn)
           + lax.dot_general(std1, wu7_ref[...], dn))
    hout = h_ref[...] + jnp.maximum(upd + bupd_ref[...], 0.0)
    hp_ref[...] = (
        lax.dot_general(wp_ref[...], hout, (((0,), (1,)), ((), ()))) + bp_ref[...]
    )


def _sqrt_body(d2_ref, out_ref):
    out_ref[...] = jnp.sqrt(d2_ref[...] + 1e-12)


def _rmw_extreme(acc_v, idxv, val, is_max):
    def cond(carry):
        return carry[0] > 0

    def body(carry):
        _, todo = carry
        cur = plsc.load_gather(acc_v, [idxv])
        new = jnp.maximum(cur, val) if is_max else jnp.minimum(cur, val)
        plsc.store_scatter(acc_v, [idxv], new, mask=todo)
        chk = plsc.load_gather(acc_v, [idxv])
        ok = (chk >= val) if is_max else (chk <= val)
        todo2 = jnp.logical_and(todo, jnp.logical_not(ok))
        return jnp.sum(todo2.astype(jnp.int32)), todo2

    lax.while_loop(cond, body, (jnp.int32(L), jnp.ones((L,), jnp.bool_)))


def _make_deg_body(n_nodes, n_edges):
    epw = n_edges // NW

    def body(dst_hbm, degp_hbm, idx_v, acc_v):
        wid = lax.axis_index("c") * NS + lax.axis_index("s")
        zz = jnp.zeros((L,), jnp.float32)

        def zloop(i, c):
            acc_v[pl.ds(i * L, L)] = zz
            return c
        lax.fori_loop(0, n_nodes // L, zloop, 0)

        pltpu.sync_copy(dst_hbm.at[pl.ds(wid * epw, epw)], idx_v)
        ones = jnp.ones((L,), jnp.float32)

        def eloop(i, c):
            plsc.addupdate_scatter(acc_v, [idx_v[pl.ds(i * L, L)]], ones)
            return c
        lax.fori_loop(0, epw // L, eloop, 0)
        pltpu.sync_copy(acc_v, degp_hbm.at[pl.ds(wid * n_nodes, n_nodes)])

    return body


def _make_edge_pass_body(pass_off, n_nodes, n_edges, chunk):
    def body(at_hbm, bt_hbm, ct_hbm, src_hbm, dst_hbm,
             sum_hbm, sq_hbm, mx_hbm, mn_hbm,
             a0_v, a1_v, b0_v, b1_v,
             s0_v, s1_v, q0_v, q1_v, x0_v, x1_v, n0_v, n1_v,
             src_b, dst_b, c0_b, c1_b, sem0, sem1, sem2, sem3):
        wid = lax.axis_index("c") * NS + lax.axis_index("s")
        f_loc = 2 * wid
        f_glob = pass_off + f_loc

        zz = jnp.zeros((L,), jnp.float32)
        lo = jnp.full((L,), FMIN, jnp.float32)
        hi = jnp.full((L,), FMAX, jnp.float32)

        def zloop(i, c):
            sl = pl.ds(i * L, L)
            s0_v[sl] = zz
            s1_v[sl] = zz
            q0_v[sl] = zz
            q1_v[sl] = zz
            x0_v[sl] = lo
            x1_v[sl] = lo
            n0_v[sl] = hi
            n1_v[sl] = hi
            return c
        lax.fori_loop(0, n_nodes // L, zloop, 0)

        nn = n_nodes
        pltpu.sync_copy(at_hbm.at[pl.ds(f_glob * nn, nn)], a0_v)
        pltpu.sync_copy(at_hbm.at[pl.ds((f_glob + 1) * nn, nn)], a1_v)
        pltpu.sync_copy(bt_hbm.at[pl.ds(f_glob * nn, nn)], b0_v)
        pltpu.sync_copy(bt_hbm.at[pl.ds((f_glob + 1) * nn, nn)], b1_v)

        def issue(ci, par):
            base = ci * chunk
            half = pl.ds(par * chunk, chunk)
            pltpu.async_copy(src_hbm.at[pl.ds(base, chunk)], src_b.at[half], sem0)
            pltpu.async_copy(dst_hbm.at[pl.ds(base, chunk)], dst_b.at[half], sem1)
            pltpu.async_copy(
                ct_hbm.at[pl.ds(f_glob * n_edges + base, chunk)], c0_b.at[half], sem2)
            pltpu.async_copy(
                ct_hbm.at[pl.ds((f_glob + 1) * n_edges + base, chunk)], c1_b.at[half], sem3)

        issue(0, 0)
        nchunks = n_edges // chunk

        def chunk_loop(ci, c):
            par = lax.rem(ci, 2)
            base = ci * chunk
            half = pl.ds(par * chunk, chunk)

            @pl.when(ci + 1 < nchunks)
            def _prefetch():
                issue(ci + 1, 1 - par)

            pltpu.make_async_copy(
                src_hbm.at[pl.ds(base, chunk)], src_b.at[half], sem0).wait()
            pltpu.make_async_copy(
                dst_hbm.at[pl.ds(base, chunk)], dst_b.at[half], sem1).wait()
            pltpu.make_async_copy(
                ct_hbm.at[pl.ds(f_glob * n_edges + base, chunk)], c0_b.at[half],
                sem2).wait()
            pltpu.make_async_copy(
                ct_hbm.at[pl.ds((f_glob + 1) * n_edges + base, chunk)], c1_b.at[half],
                sem3).wait()

            def prep(off):
                sl = pl.ds(par * chunk + off, L)
                srcv = src_b[sl]
                dstv = dst_b[sl]
                cnt, _ = plsc.scan_count(dstv)
                neq = plsc.all_reduce_population_count(cnt == cnt[0])
                dup = neq[0] < L
                m0 = jnp.maximum(
                    plsc.load_gather(a0_v, [srcv])
                    + plsc.load_gather(b0_v, [dstv]) + c0_b[sl], 0.0)
                m1 = jnp.maximum(
                    plsc.load_gather(a1_v, [srcv])
                    + plsc.load_gather(b1_v, [dstv]) + c1_b[sl], 0.0)
                plsc.addupdate_scatter(s0_v, [dstv], m0)
                plsc.addupdate_scatter(q0_v, [dstv], m0 * m0)
                plsc.addupdate_scatter(s1_v, [dstv], m1)
                plsc.addupdate_scatter(q1_v, [dstv], m1 * m1)
                return dstv, m0, m1, dup

            def rmw(dstv, m0, m1, dup):
                def _fast():
                    cx0 = plsc.load_gather(x0_v, [dstv])
                    cn0 = plsc.load_gather(n0_v, [dstv])
                    cx1 = plsc.load_gather(x1_v, [dstv])
                    cn1 = plsc.load_gather(n1_v, [dstv])
                    plsc.store_scatter(x0_v, [dstv], jnp.maximum(cx0, m0))
                    plsc.store_scatter(n0_v, [dstv], jnp.minimum(cn0, m0))
                    plsc.store_scatter(x1_v, [dstv], jnp.maximum(cx1, m1))
                    plsc.store_scatter(n1_v, [dstv], jnp.minimum(cn1, m1))

                _fast()

                @pl.when(dup)
                def _slow():
                    _rmw_extreme(x0_v, dstv, m0, True)
                    _rmw_extreme(n0_v, dstv, m0, False)
                    _rmw_extreme(x1_v, dstv, m1, True)
                    _rmw_extreme(n1_v, dstv, m1, False)

            def vec_loop(i, cc):
                ts = [prep(i * 4 * L + k * L) for k in range(4)]
                for t in ts:
                    rmw(*t)
                return cc
            lax.fori_loop(0, chunk // (4 * L), vec_loop, 0)
            return c
        lax.fori_loop(0, nchunks, chunk_loop, 0)

        pltpu.sync_copy(s0_v, sum_hbm.at[pl.ds(f_loc * nn, nn)])
        pltpu.sync_copy(s1_v, sum_hbm.at[pl.ds((f_loc + 1) * nn, nn)])
        pltpu.sync_copy(q0_v, sq_hbm.at[pl.ds(f_loc * nn, nn)])
        pltpu.sync_copy(q1_v, sq_hbm.at[pl.ds((f_loc + 1) * nn, nn)])
        pltpu.sync_copy(x0_v, mx_hbm.at[pl.ds(f_loc * nn, nn)])
        pltpu.sync_copy(x1_v, mx_hbm.at[pl.ds((f_loc + 1) * nn, nn)])
        pltpu.sync_copy(n0_v, mn_hbm.at[pl.ds(f_loc * nn, nn)])
        pltpu.sync_copy(n1_v, mn_hbm.at[pl.ds((f_loc + 1) * nn, nn)])

    return body


def _make_pair_body(n_nodes, n_pairs):
    ppw = n_pairs // NW

    def body(hp_hbm, pi_hbm, d2_hbm, hx_v, hy_v, hz_v, i0_v, i1_v, out_v):
        wid = lax.axis_index("c") * NS + lax.axis_index("s")
        base = wid * ppw
        nn = n_nodes
        pltpu.sync_copy(hp_hbm.at[pl.ds(0, nn)], hx_v)
        pltpu.sync_copy(hp_hbm.at[pl.ds(nn, nn)], hy_v)
        pltpu.sync_copy(hp_hbm.at[pl.ds(2 * nn, nn)], hz_v)
        pltpu.sync_copy(pi_hbm.at[pl.ds(base, ppw)], i0_v)
        pltpu.sync_copy(pi_hbm.at[pl.ds(n_pairs + base, ppw)], i1_v)

        def loop(i, c):
            sl = pl.ds(i * L, L)
            i0 = i0_v[sl]
            i1 = i1_v[sl]
            dx = plsc.load_gather(hx_v, [i0]) - plsc.load_gather(hx_v, [i1])
            dy = plsc.load_gather(hy_v, [i0]) - plsc.load_gather(hy_v, [i1])
            dz = plsc.load_gather(hz_v, [i0]) - plsc.load_gather(hz_v, [i1])
            out_v[sl] = dx * dx + dy * dy + dz * dz
            return c
        lax.fori_loop(0, ppw // L, loop, 0)
        pltpu.sync_copy(out_v, d2_hbm.at[pl.ds(base, ppw)])

    return body


_sc_mesh_cache = []


def _sc_mesh():
    if not _sc_mesh_cache:
        _sc_mesh_cache.append(plsc.VectorSubcoreMesh(
            core_axis_name="c", subcore_axis_name="s",
            num_cores=NC, num_subcores=NS,
        ))
    return _sc_mesh_cache[0]


def _full(shape):
    return pl.BlockSpec(shape, lambda i: (0, 0))


def kernel(x, pos, edge_index, edge_attr, pairwise_indices,
           W_in, b_in, W_msg, b_msg, W_upd, b_upd, W_proj, b_proj):
    n, d = x.shape
    e, ed = edge_attr.shape
    h_dim = W_in.shape[1]
    p = pairwise_indices.shape[1]
    del pos

    nb = 1024
    eb = 6400
    chunk = 640
    n_pad = ((n + nb - 1) // nb) * nb
    x = jnp.pad(x, ((0, n_pad - n), (0, 0)))

    f32 = jnp.float32
    w1 = W_msg[:h_dim]
    w2 = W_msg[h_dim:2 * h_dim]
    w3 = W_msg[2 * h_dim:]
    ea_t = edge_attr.T
    bin2 = b_in.reshape(1, h_dim)
    bmsg2 = b_msg.reshape(h_dim, 1)
    bupd2 = b_upd.reshape(1, h_dim)
    wp8 = jnp.pad(W_proj, ((0, 0), (0, 8 - W_proj.shape[1])))
    bp8 = jnp.pad(b_proj, (0, 8 - b_proj.shape[0])).reshape(8, 1)

    h, a_t, b_t = pl.pallas_call(
        _node_pre_body,
        grid=(n_pad // nb,),
        in_specs=[
            pl.BlockSpec((nb, d), lambda i: (i, 0)),
            _full((d, h_dim)), _full((1, h_dim)),
            _full((h_dim, h_dim)), _full((h_dim, h_dim)),
        ],
        out_specs=[
            pl.BlockSpec((nb, h_dim), lambda i: (i, 0)),
            pl.BlockSpec((h_dim, nb), lambda i: (0, i)),
            pl.BlockSpec((h_dim, nb), lambda i: (0, i)),
        ],
        out_shape=[
            jax.ShapeDtypeStruct((n_pad, h_dim), f32),
            jax.ShapeDtypeStruct((h_dim, n_pad), f32),
            jax.ShapeDtypeStruct((h_dim, n_pad), f32),
        ],
    )(x, W_in, bin2, w1, w2)

    c_t = pl.pallas_call(
        _edge_c_body,
        grid=(e // eb,),
        in_specs=[
            pl.BlockSpec((ed, eb), lambda i: (0, i)),
            _full((ed, h_dim)), _full((h_dim, 1)),
        ],
        out_specs=pl.BlockSpec((h_dim, eb), lambda i: (0, i)),
        out_shape=jax.ShapeDtypeStruct((h_dim, e), f32),
    )(ea_t, w3, bmsg2)

    src = edge_index[0]
    dst = edge_index[1]

    degp = pl.kernel(
        _make_deg_body(n_pad, e),
        compiler_params=pltpu.CompilerParams(needs_layout_passes=False),
        out_type=jax.ShapeDtypeStruct((NW * n_pad,), f32),
        mesh=_sc_mesh(),
        scratch_types=[
            pltpu.VMEM((e // NW,), jnp.int32),
            pltpu.VMEM((n_pad,), f32),
        ],
    )(dst)
    degp = degp.reshape(NW, n_pad)

    def run_pass(pass_off):
        return pl.kernel(
            _make_edge_pass_body(pass_off, n_pad, e, chunk),
            compiler_params=pltpu.CompilerParams(needs_layout_passes=False),
            out_type=[jax.ShapeDtypeStruct((h_dim // 2 * n_pad,), f32)] * 4,
            mesh=_sc_mesh(),
            scratch_types=(
                [pltpu.VMEM((n_pad,), f32)] * 4
                + [pltpu.VMEM((n_pad,), f32)] * 8
                + [pltpu.VMEM((2 * chunk,), jnp.int32)] * 2
                + [pltpu.VMEM((2 * chunk,), f32)] * 2
                + [pltpu.SemaphoreType.DMA] * 4
            ),
        )(a_t.reshape(-1), b_t.reshape(-1), c_t.reshape(-1), src, dst)

    hw = h_dim // 2
    s0, q0, x0, n0 = (r.reshape(hw, n_pad) for r in run_pass(0))
    s1, q1, x1, n1 = (r.reshape(hw, n_pad) for r in run_pass(hw))

    wu = [W_upd[i * hw:(i + 1) * hw] for i in range(8)]
    half = pl.BlockSpec((hw, nb), lambda i: (0, i))
    hp = pl.pallas_call(
        _finalize_body,
        grid=(n_pad // nb,),
        in_specs=(
            [half] * 8
            + [pl.BlockSpec((NW, nb), lambda i: (0, i)),
               pl.BlockSpec((nb, h_dim), lambda i: (i, 0))]
            + [_full((hw, h_dim))] * 8
            + [_full((1, h_dim)), _full((h_dim, 8)), _full((8, 1))]
        ),
        out_specs=pl.BlockSpec((8, nb), lambda i: (0, i)),
        out_shape=jax.ShapeDtypeStruct((8, n_pad), f32),
    )(s0, s1, q0, q1, x0, x1, n0, n1, degp, h, *wu, bupd2, wp8, bp8)

    d2 = pl.kernel(
        _make_pair_body(n_pad, p),
        compiler_params=pltpu.CompilerParams(needs_layout_passes=False),
        out_type=jax.ShapeDtypeStruct((p,), f32),
        mesh=_sc_mesh(),
        scratch_types=(
            [pltpu.VMEM((n_pad,), f32)] * 3
            + [pltpu.VMEM((p // NW,), jnp.int32)] * 2
            + [pltpu.VMEM((p // NW,), f32)]
        ),
    )(hp.reshape(-1), pairwise_indices.reshape(-1))

    rows = 2500
    dist = pl.pallas_call(
        _sqrt_body,
        out_shape=jax.ShapeDtypeStruct((rows, p // rows), f32),
    )(d2.reshape(rows, p // rows))

    return dist.reshape(p, 1)


# speedup vs baseline: 17.3355x; 1.0173x over previous
import jax
import jax.numpy as jnp
from jax import lax
from jax.experimental import pallas as pl
from jax.experimental.pallas import tpu as pltpu
from jax.experimental.pallas import tpu_sc as plsc

NC = 2
NS = 16
NW = NC * NS
L = 16

FMIN = -3.4e38
FMAX = 3.4e38


def _node_pre_body(x_ref, win_ref, bin_ref, w1_ref, w2_ref, h_ref, at_ref, bt_ref):
    h = jnp.maximum(jnp.dot(x_ref[...], win_ref[...]) + bin_ref[...], 0.0)
    h_ref[...] = h
    at_ref[...] = lax.dot_general(w1_ref[...], h, (((0,), (1,)), ((), ())))
    bt_ref[...] = lax.dot_general(w2_ref[...], h, (((0,), (1,)), ((), ())))


def _edge_c_body(eat_ref, w3_ref, bmsg_ref, ct_ref):
    ct_ref[...] = (
        lax.dot_general(w3_ref[...], eat_ref[...], (((0,), (0,)), ((), ())))
        + bmsg_ref[...]
    )


def _finalize_body(s0_ref, s1_ref, q0_ref, q1_ref, x0_ref, x1_ref, n0_ref, n1_ref,
                   degp_ref, h_ref,
                   wu0_ref, wu1_ref, wu2_ref, wu3_ref, wu4_ref, wu5_ref, wu6_ref,
                   wu7_ref, bupd_ref, wp_ref, bp_ref, hp_ref):
    deg = jnp.sum(degp_ref[...], axis=0, keepdims=True)
    rden = 1.0 / jnp.maximum(deg, 1.0)
    nonempty = deg > 0.0

    def stats(s_ref, q_ref, x_ref, n_ref):
        mean = s_ref[...] * rden
        var = jnp.maximum(q_ref[...] * rden - mean * mean, 0.0)
        std = jnp.sqrt(var + 1e-5)
        mx = jnp.where(nonempty, x_ref[...], 0.0)
        mn = jnp.where(nonempty, n_ref[...], 0.0)
        return mean, mx, mn, std

    mean0, mx0, mn0, std0 = stats(s0_ref, q0_ref, x0_ref, n0_ref)
    mean1, mx1, mn1, std1 = stats(s1_ref, q1_ref, x1_ref, n1_ref)

    dn = (((0,), (0,)), ((), ()))
    upd = (lax.dot_general(mean0, wu0_ref[...], dn)
           + lax.dot_general(mean1, wu1_ref[...], dn)
           + lax.dot_general(mx0, wu2_ref[...], dn)
           + lax.dot_general(mx1, wu3_ref[...], dn)
           + lax.dot_general(mn0, wu4_ref[...], dn)
           + lax.dot_general(mn1, wu5_ref[...], dn)
           + lax.dot_general(std0, wu6_ref[...], dn)
           + lax.dot_general(std1, wu7_ref[...], dn))
    hout = h_ref[...] + jnp.maximum(upd + bupd_ref[...], 0.0)
    hp_ref[...] = (
        lax.dot_general(wp_ref[...], hout, (((0,), (1,)), ((), ()))) + bp_ref[...]
    )


def _sqrt_body(d2_ref, out_ref):
    out_ref[...] = jnp.sqrt(d2_ref[...] + 1e-12)


def _rmw_extreme(acc_v, idxv, val, is_max):
    def cond(carry):
        return carry[0] > 0

    def body(carry):
        _, todo = carry
        cur = plsc.load_gather(acc_v, [idxv])
        new = jnp.maximum(cur, val) if is_max else jnp.minimum(cur, val)
        plsc.store_scatter(acc_v, [idxv], new, mask=todo)
        chk = plsc.load_gather(acc_v, [idxv])
        ok = (chk >= val) if is_max else (chk <= val)
        todo2 = jnp.logical_and(todo, jnp.logical_not(ok))
        return jnp.sum(todo2.astype(jnp.int32)), todo2

    lax.while_loop(cond, body, (jnp.int32(L), jnp.ones((L,), jnp.bool_)))


def _make_deg_body(n_nodes, n_edges):
    epw = n_edges // NW

    def body(dst_hbm, degp_hbm, idx_v, acc_v):
        wid = lax.axis_index("c") * NS + lax.axis_index("s")
        zz = jnp.zeros((L,), jnp.float32)

        def zloop(i, c):
            acc_v[pl.ds(i * L, L)] = zz
            return c
        lax.fori_loop(0, n_nodes // L, zloop, 0)

        pltpu.sync_copy(dst_hbm.at[pl.ds(wid * epw, epw)], idx_v)
        ones = jnp.ones((L,), jnp.float32)

        def eloop(i, c):
            plsc.addupdate_scatter(acc_v, [idx_v[pl.ds(i * L, L)]], ones)
            return c
        lax.fori_loop(0, epw // L, eloop, 0)
        pltpu.sync_copy(acc_v, degp_hbm.at[pl.ds(wid * n_nodes, n_nodes)])

    return body


def _make_edge_pass_body(pass_off, n_nodes, n_edges, chunk):
    def body(at_hbm, bt_hbm, ct_hbm, src_hbm, dst_hbm,
             sum_hbm, sq_hbm, mx_hbm, mn_hbm,
             a0_v, a1_v, b0_v, b1_v,
             s0_v, s1_v, q0_v, q1_v, x0_v, x1_v, n0_v, n1_v,
             src_b, dst_b, c0_b, c1_b, sem0, sem1, sem2, sem3):
        wid = lax.axis_index("c") * NS + lax.axis_index("s")
        f_loc = 2 * wid
        f_glob = pass_off + f_loc

        zz = jnp.zeros((L,), jnp.float32)
        lo = jnp.full((L,), FMIN, jnp.float32)
        hi = jnp.full((L,), FMAX, jnp.float32)

        def zloop(i, c):
            sl = pl.ds(i * L, L)
            s0_v[sl] = zz
            s1_v[sl] = zz
            q0_v[sl] = zz
            q1_v[sl] = zz
            x0_v[sl] = lo
            x1_v[sl] = lo
            n0_v[sl] = hi
            n1_v[sl] = hi
            return c
        lax.fori_loop(0, n_nodes // L, zloop, 0)

        nn = n_nodes
        pltpu.sync_copy(at_hbm.at[pl.ds(f_glob * nn, nn)], a0_v)
        pltpu.sync_copy(at_hbm.at[pl.ds((f_glob + 1) * nn, nn)], a1_v)
        pltpu.sync_copy(bt_hbm.at[pl.ds(f_glob * nn, nn)], b0_v)
        pltpu.sync_copy(bt_hbm.at[pl.ds((f_glob + 1) * nn, nn)], b1_v)

        def issue(ci, par):
            base = ci * chunk
            half = pl.ds(par * chunk, chunk)
            pltpu.async_copy(src_hbm.at[pl.ds(base, chunk)], src_b.at[half], sem0)
            pltpu.async_copy(dst_hbm.at[pl.ds(base, chunk)], dst_b.at[half], sem1)
            pltpu.async_copy(
                ct_hbm.at[pl.ds(f_glob * n_edges + base, chunk)], c0_b.at[half], sem2)
            pltpu.async_copy(
                ct_hbm.at[pl.ds((f_glob + 1) * n_edges + base, chunk)], c1_b.at[half], sem3)

        issue(0, 0)
        nchunks = n_edges // chunk

        def chunk_loop(ci, c):
            par = lax.rem(ci, 2)
            base = ci * chunk
            half = pl.ds(par * chunk, chunk)

            @pl.when(ci + 1 < nchunks)
            def _prefetch():
                issue(ci + 1, 1 - par)

            pltpu.make_async_copy(
                src_hbm.at[pl.ds(base, chunk)], src_b.at[half], sem0).wait()
            pltpu.make_async_copy(
                dst_hbm.at[pl.ds(base, chunk)], dst_b.at[half], sem1).wait()
            pltpu.make_async_copy(
                ct_hbm.at[pl.ds(f_glob * n_edges + base, chunk)], c0_b.at[half],
                sem2).wait()
            pltpu.make_async_copy(
                ct_hbm.at[pl.ds((f_glob + 1) * n_edges + base, chunk)], c1_b.at[half],
                sem3).wait()

            def prep(off):
                sl = pl.ds(par * chunk + off, L)
                srcv = src_b[sl]
                dstv = dst_b[sl]
                cnt, _ = plsc.scan_count(dstv)
                neq = plsc.all_reduce_population_count(cnt == cnt[0])
                dup = neq[0] < L
                m0 = jnp.maximum(
                    plsc.load_gather(a0_v, [srcv])
                    + plsc.load_gather(b0_v, [dstv]) + c0_b[sl], 0.0)
                m1 = jnp.maximum(
                    plsc.load_gather(a1_v, [srcv])
                    + plsc.load_gather(b1_v, [dstv]) + c1_b[sl], 0.0)
                plsc.addupdate_scatter(s0_v, [dstv], m0)
                plsc.addupdate_scatter(q0_v, [dstv], m0 * m0)
                plsc.addupdate_scatter(s1_v, [dstv], m1)
                plsc.addupdate_scatter(q1_v, [dstv], m1 * m1)
                return dstv, m0, m1, dup

            def rmw(dstv, m0, m1, dup):
                def _fast():
                    cx0 = plsc.load_gather(x0_v, [dstv])
                    cn0 = plsc.load_gather(n0_v, [dstv])
                    cx1 = plsc.load_gather(x1_v, [dstv])
                    cn1 = plsc.load_gather(n1_v, [dstv])
                    plsc.store_scatter(x0_v, [dstv], jnp.maximum(cx0, m0))
                    plsc.store_scatter(n0_v, [dstv], jnp.minimum(cn0, m0))
                    plsc.store_scatter(x1_v, [dstv], jnp.maximum(cx1, m1))
                    plsc.store_scatter(n1_v, [dstv], jnp.minimum(cn1, m1))

                _fast()

                @pl.when(dup)
                def _slow():
                    _rmw_extreme(x0_v, dstv, m0, True)
                    _rmw_extreme(n0_v, dstv, m0, False)
                    _rmw_extreme(x1_v, dstv, m1, True)
                    _rmw_extreme(n1_v, dstv, m1, False)

            def vec_loop(i, cc):
                ts = [prep(i * 8 * L + k * L) for k in range(8)]
                for t in ts:
                    rmw(*t)
                return cc
            lax.fori_loop(0, chunk // (8 * L), vec_loop, 0)
            return c
        lax.fori_loop(0, nchunks, chunk_loop, 0)

        pltpu.sync_copy(s0_v, sum_hbm.at[pl.ds(f_loc * nn, nn)])
        pltpu.sync_copy(s1_v, sum_hbm.at[pl.ds((f_loc + 1) * nn, nn)])
        pltpu.sync_copy(q0_v, sq_hbm.at[pl.ds(f_loc * nn, nn)])
        pltpu.sync_copy(q1_v, sq_hbm.at[pl.ds((f_loc + 1) * nn, nn)])
        pltpu.sync_copy(x0_v, mx_hbm.at[pl.ds(f_loc * nn, nn)])
        pltpu.sync_copy(x1_v, mx_hbm.at[pl.ds((f_loc + 1) * nn, nn)])
        pltpu.sync_copy(n0_v, mn_hbm.at[pl.ds(f_loc * nn, nn)])
        pltpu.sync_copy(n1_v, mn_hbm.at[pl.ds((f_loc + 1) * nn, nn)])

    return body


def _make_pair_body(n_nodes, n_pairs):
    ppw = n_pairs // NW

    def body(hp_hbm, pi_hbm, d2_hbm, hx_v, hy_v, hz_v, i0_v, i1_v, out_v):
        wid = lax.axis_index("c") * NS + lax.axis_index("s")
        base = wid * ppw
        nn = n_nodes
        pltpu.sync_copy(hp_hbm.at[pl.ds(0, nn)], hx_v)
        pltpu.sync_copy(hp_hbm.at[pl.ds(nn, nn)], hy_v)
        pltpu.sync_copy(hp_hbm.at[pl.ds(2 * nn, nn)], hz_v)
        pltpu.sync_copy(pi_hbm.at[pl.ds(base, ppw)], i0_v)
        pltpu.sync_copy(pi_hbm.at[pl.ds(n_pairs + base, ppw)], i1_v)

        def loop(i, c):
            sl = pl.ds(i * L, L)
            i0 = i0_v[sl]
            i1 = i1_v[sl]
            dx = plsc.load_gather(hx_v, [i0]) - plsc.load_gather(hx_v, [i1])
            dy = plsc.load_gather(hy_v, [i0]) - plsc.load_gather(hy_v, [i1])
            dz = plsc.load_gather(hz_v, [i0]) - plsc.load_gather(hz_v, [i1])
            out_v[sl] = dx * dx + dy * dy + dz * dz
            return c
        lax.fori_loop(0, ppw // L, loop, 0)
        pltpu.sync_copy(out_v, d2_hbm.at[pl.ds(base, ppw)])

    return body


_sc_mesh_cache = []


def _sc_mesh():
    if not _sc_mesh_cache:
        _sc_mesh_cache.append(plsc.VectorSubcoreMesh(
            core_axis_name="c", subcore_axis_name="s",
            num_cores=NC, num_subcores=NS,
        ))
    return _sc_mesh_cache[0]


def _full(shape):
    return pl.BlockSpec(shape, lambda i: (0, 0))


def kernel(x, pos, edge_index, edge_attr, pairwise_indices,
           W_in, b_in, W_msg, b_msg, W_upd, b_upd, W_proj, b_proj):
    n, d = x.shape
    e, ed = edge_attr.shape
    h_dim = W_in.shape[1]
    p = pairwise_indices.shape[1]
    del pos

    nb = 1024
    eb = 6400
    chunk = 640
    n_pad = ((n + nb - 1) // nb) * nb
    x = jnp.pad(x, ((0, n_pad - n), (0, 0)))

    f32 = jnp.float32
    w1 = W_msg[:h_dim]
    w2 = W_msg[h_dim:2 * h_dim]
    w3 = W_msg[2 * h_dim:]
    ea_t = edge_attr.T
    bin2 = b_in.reshape(1, h_dim)
    bmsg2 = b_msg.reshape(h_dim, 1)
    bupd2 = b_upd.reshape(1, h_dim)
    wp8 = jnp.pad(W_proj, ((0, 0), (0, 8 - W_proj.shape[1])))
    bp8 = jnp.pad(b_proj, (0, 8 - b_proj.shape[0])).reshape(8, 1)

    h, a_t, b_t = pl.pallas_call(
        _node_pre_body,
        grid=(n_pad // nb,),
        in_specs=[
            pl.BlockSpec((nb, d), lambda i: (i, 0)),
            _full((d, h_dim)), _full((1, h_dim)),
            _full((h_dim, h_dim)), _full((h_dim, h_dim)),
        ],
        out_specs=[
            pl.BlockSpec((nb, h_dim), lambda i: (i, 0)),
            pl.BlockSpec((h_dim, nb), lambda i: (0, i)),
            pl.BlockSpec((h_dim, nb), lambda i: (0, i)),
        ],
        out_shape=[
            jax.ShapeDtypeStruct((n_pad, h_dim), f32),
            jax.ShapeDtypeStruct((h_dim, n_pad), f32),
            jax.ShapeDtypeStruct((h_dim, n_pad), f32),
        ],
    )(x, W_in, bin2, w1, w2)

    c_t = pl.pallas_call(
        _edge_c_body,
        grid=(e // eb,),
        in_specs=[
            pl.BlockSpec((ed, eb), lambda i: (0, i)),
            _full((ed, h_dim)), _full((h_dim, 1)),
        ],
        out_specs=pl.BlockSpec((h_dim, eb), lambda i: (0, i)),
        out_shape=jax.ShapeDtypeStruct((h_dim, e), f32),
    )(ea_t, w3, bmsg2)

    src = edge_index[0]
    dst = edge_index[1]

    degp = pl.kernel(
        _make_deg_body(n_pad, e),
        compiler_params=pltpu.CompilerParams(needs_layout_passes=False),
        out_type=jax.ShapeDtypeStruct((NW * n_pad,), f32),
        mesh=_sc_mesh(),
        scratch_types=[
            pltpu.VMEM((e // NW,), jnp.int32),
            pltpu.VMEM((n_pad,), f32),
        ],
    )(dst)
    degp = degp.reshape(NW, n_pad)

    def run_pass(pass_off):
        return pl.kernel(
            _make_edge_pass_body(pass_off, n_pad, e, chunk),
            compiler_params=pltpu.CompilerParams(needs_layout_passes=False),
            out_type=[jax.ShapeDtypeStruct((h_dim // 2 * n_pad,), f32)] * 4,
            mesh=_sc_mesh(),
            scratch_types=(
                [pltpu.VMEM((n_pad,), f32)] * 4
                + [pltpu.VMEM((n_pad,), f32)] * 8
                + [pltpu.VMEM((2 * chunk,), jnp.int32)] * 2
                + [pltpu.VMEM((2 * chunk,), f32)] * 2
                + [pltpu.SemaphoreType.DMA] * 4
            ),
        )(a_t.reshape(-1), b_t.reshape(-1), c_t.reshape(-1), src, dst)

    hw = h_dim // 2
    s0, q0, x0, n0 = (r.reshape(hw, n_pad) for r in run_pass(0))
    s1, q1, x1, n1 = (r.reshape(hw, n_pad) for r in run_pass(hw))

    wu = [W_upd[i * hw:(i + 1) * hw] for i in range(8)]
    half = pl.BlockSpec((hw, nb), lambda i: (0, i))
    hp = pl.pallas_call(
        _finalize_body,
        grid=(n_pad // nb,),
        in_specs=(
            [half] * 8
            + [pl.BlockSpec((NW, nb), lambda i: (0, i)),
               pl.BlockSpec((nb, h_dim), lambda i: (i, 0))]
            + [_full((hw, h_dim))] * 8
            + [_full((1, h_dim)), _full((h_dim, 8)), _full((8, 1))]
        ),
        out_specs=pl.BlockSpec((8, nb), lambda i: (0, i)),
        out_shape=jax.ShapeDtypeStruct((8, n_pad), f32),
    )(s0, s1, q0, q1, x0, x1, n0, n1, degp, h, *wu, bupd2, wp8, bp8)

    d2 = pl.kernel(
        _make_pair_body(n_pad, p),
        compiler_params=pltpu.CompilerParams(needs_layout_passes=False),
        out_type=jax.ShapeDtypeStruct((p,), f32),
        mesh=_sc_mesh(),
        scratch_types=(
            [pltpu.VMEM((n_pad,), f32)] * 3
            + [pltpu.VMEM((p // NW,), jnp.int32)] * 2
            + [pltpu.VMEM((p // NW,), f32)]
        ),
    )(hp.reshape(-1), pairwise_indices.reshape(-1))

    rows = 2500
    dist = pl.pallas_call(
        _sqrt_body,
        out_shape=jax.ShapeDtypeStruct((rows, p // rows), f32),
    )(d2.reshape(rows, p // rows))

    return dist.reshape(p, 1)


# speedup vs baseline: 17.3819x; 1.0027x over previous
import jax
import jax.numpy as jnp
from jax import lax
from jax.experimental import pallas as pl
from jax.experimental.pallas import tpu as pltpu
from jax.experimental.pallas import tpu_sc as plsc

NC = 2
NS = 16
NW = NC * NS
L = 16

FMIN = -3.4e38
FMAX = 3.4e38


def _node_pre_body(x_ref, win_ref, bin_ref, w1_ref, w2_ref, h_ref, at_ref, bt_ref):
    h = jnp.maximum(jnp.dot(x_ref[...], win_ref[...]) + bin_ref[...], 0.0)
    h_ref[...] = h
    at_ref[...] = lax.dot_general(w1_ref[...], h, (((0,), (1,)), ((), ())))
    bt_ref[...] = lax.dot_general(w2_ref[...], h, (((0,), (1,)), ((), ())))


def _edge_c_body(eat_ref, w3_ref, bmsg_ref, ct_ref):
    ct_ref[...] = (
        lax.dot_general(w3_ref[...], eat_ref[...], (((0,), (0,)), ((), ())))
        + bmsg_ref[...]
    )


def _finalize_body(s_ref, q_ref, x_ref, n_ref, degp_ref, h_ref,
                   wum_ref, wux_ref, wun_ref, wus_ref,
                   bupd_ref, wp_ref, bp_ref, hp_ref):
    deg = jnp.sum(degp_ref[...], axis=0, keepdims=True)
    rden = 1.0 / jnp.maximum(deg, 1.0)
    nonempty = deg > 0.0
    mean = s_ref[...] * rden
    var = jnp.maximum(q_ref[...] * rden - mean * mean, 0.0)
    std = jnp.sqrt(var + 1e-5)
    mx = jnp.where(nonempty, x_ref[...], 0.0)
    mn = jnp.where(nonempty, n_ref[...], 0.0)
    dn = (((0,), (0,)), ((), ()))
    upd = (lax.dot_general(mean, wum_ref[...], dn)
           + lax.dot_general(mx, wux_ref[...], dn)
           + lax.dot_general(mn, wun_ref[...], dn)
           + lax.dot_general(std, wus_ref[...], dn))
    hout = h_ref[...] + jnp.maximum(upd + bupd_ref[...], 0.0)
    hp_ref[...] = (
        lax.dot_general(wp_ref[...], hout, (((0,), (1,)), ((), ()))) + bp_ref[...]
    )


def _sqrt_body(d2_ref, out_ref):
    out_ref[...] = jnp.sqrt(d2_ref[...] + 1e-12)


def _rmw_extreme(acc_v, idxv, val, is_max):
    def cond(carry):
        return carry[0] > 0

    def body(carry):
        _, todo = carry
        cur = plsc.load_gather(acc_v, [idxv])
        new = jnp.maximum(cur, val) if is_max else jnp.minimum(cur, val)
        plsc.store_scatter(acc_v, [idxv], new, mask=todo)
        chk = plsc.load_gather(acc_v, [idxv])
        ok = (chk >= val) if is_max else (chk <= val)
        todo2 = jnp.logical_and(todo, jnp.logical_not(ok))
        return jnp.sum(todo2.astype(jnp.int32)), todo2

    lax.while_loop(cond, body, (jnp.int32(L), jnp.ones((L,), jnp.bool_)))


def _make_deg_body(n_nodes, n_edges):
    epw = n_edges // NW

    def body(dst_hbm, degp_hbm, idx_v, acc_v):
        wid = lax.axis_index("c") * NS + lax.axis_index("s")
        zz = jnp.zeros((L,), jnp.float32)

        def zloop(i, c):
            acc_v[pl.ds(i * L, L)] = zz
            return c
        lax.fori_loop(0, n_nodes // L, zloop, 0)

        pltpu.sync_copy(dst_hbm.at[pl.ds(wid * epw, epw)], idx_v)
        ones = jnp.ones((L,), jnp.float32)

        def eloop(i, c):
            plsc.addupdate_scatter(acc_v, [idx_v[pl.ds(i * L, L)]], ones)
            return c
        lax.fori_loop(0, epw // L, eloop, 0)
        pltpu.sync_copy(acc_v, degp_hbm.at[pl.ds(wid * n_nodes, n_nodes)])

    return body


def _make_edge_body(half, n_nodes, n_edges, chunk):
    def body(at_hbm, bt_hbm, ct_hbm, src_hbm, dst_hbm,
             sum_hbm, sq_hbm, mx_hbm, mn_hbm,
             a0_v, a1_v, b0_v, b1_v,
             s0_v, s1_v, q0_v, q1_v, x0_v, x1_v, n0_v, n1_v,
             src_b, dst_b, c0_b, c1_b, sem0, sem1, sem2, sem3):
        wid = lax.axis_index("c") * NS + lax.axis_index("s")
        for p in range(2):
          if True:
            f_loc = p * half + 2 * wid
            f_glob = f_loc

          zz = jnp.zeros((L,), jnp.float32)
          lo = jnp.full((L,), FMIN, jnp.float32)
          hi = jnp.full((L,), FMAX, jnp.float32)

          def zloop(i, c):
              sl = pl.ds(i * L, L)
              s0_v[sl] = zz
              s1_v[sl] = zz
              q0_v[sl] = zz
              q1_v[sl] = zz
              x0_v[sl] = lo
              x1_v[sl] = lo
              n0_v[sl] = hi
              n1_v[sl] = hi
              return c
          lax.fori_loop(0, n_nodes // L, zloop, 0)

          nn = n_nodes
          pltpu.sync_copy(at_hbm.at[pl.ds(f_glob * nn, nn)], a0_v)
          pltpu.sync_copy(at_hbm.at[pl.ds((f_glob + 1) * nn, nn)], a1_v)
          pltpu.sync_copy(bt_hbm.at[pl.ds(f_glob * nn, nn)], b0_v)
          pltpu.sync_copy(bt_hbm.at[pl.ds((f_glob + 1) * nn, nn)], b1_v)

          def issue(ci, par):
              base = ci * chunk
              half = pl.ds(par * chunk, chunk)
              pltpu.async_copy(src_hbm.at[pl.ds(base, chunk)], src_b.at[half], sem0)
              pltpu.async_copy(dst_hbm.at[pl.ds(base, chunk)], dst_b.at[half], sem1)
              pltpu.async_copy(
                  ct_hbm.at[pl.ds(f_glob * n_edges + base, chunk)], c0_b.at[half], sem2)
              pltpu.async_copy(
                  ct_hbm.at[pl.ds((f_glob + 1) * n_edges + base, chunk)], c1_b.at[half], sem3)

          issue(0, 0)
          nchunks = n_edges // chunk

          def chunk_loop(ci, c):
              par = lax.rem(ci, 2)
              base = ci * chunk
              half = pl.ds(par * chunk, chunk)

              @pl.when(ci + 1 < nchunks)
              def _prefetch():
                  issue(ci + 1, 1 - par)

              pltpu.make_async_copy(
                  src_hbm.at[pl.ds(base, chunk)], src_b.at[half], sem0).wait()
              pltpu.make_async_copy(
                  dst_hbm.at[pl.ds(base, chunk)], dst_b.at[half], sem1).wait()
              pltpu.make_async_copy(
                  ct_hbm.at[pl.ds(f_glob * n_edges + base, chunk)], c0_b.at[half],
                  sem2).wait()
              pltpu.make_async_copy(
                  ct_hbm.at[pl.ds((f_glob + 1) * n_edges + base, chunk)], c1_b.at[half],
                  sem3).wait()

              def prep(off):
                  sl = pl.ds(par * chunk + off, L)
                  srcv = src_b[sl]
                  dstv = dst_b[sl]
                  cnt, _ = plsc.scan_count(dstv)
                  neq = plsc.all_reduce_population_count(cnt == cnt[0])
                  dup = neq[0] < L
                  m0 = jnp.maximum(
                      plsc.load_gather(a0_v, [srcv])
                      + plsc.load_gather(b0_v, [dstv]) + c0_b[sl], 0.0)
                  m1 = jnp.maximum(
                      plsc.load_gather(a1_v, [srcv])
                      + plsc.load_gather(b1_v, [dstv]) + c1_b[sl], 0.0)
                  plsc.addupdate_scatter(s0_v, [dstv], m0)
                  plsc.addupdate_scatter(q0_v, [dstv], m0 * m0)
                  plsc.addupdate_scatter(s1_v, [dstv], m1)
                  plsc.addupdate_scatter(q1_v, [dstv], m1 * m1)
                  return dstv, m0, m1, dup

              def rmw(dstv, m0, m1, dup):
                  def _fast():
                      cx0 = plsc.load_gather(x0_v, [dstv])
                      cn0 = plsc.load_gather(n0_v, [dstv])
                      cx1 = plsc.load_gather(x1_v, [dstv])
                      cn1 = plsc.load_gather(n1_v, [dstv])
                      plsc.store_scatter(x0_v, [dstv], jnp.maximum(cx0, m0))
                      plsc.store_scatter(n0_v, [dstv], jnp.minimum(cn0, m0))
                      plsc.store_scatter(x1_v, [dstv], jnp.maximum(cx1, m1))
                      plsc.store_scatter(n1_v, [dstv], jnp.minimum(cn1, m1))

                  _fast()

                  @pl.when(dup)
                  def _slow():
                      _rmw_extreme(x0_v, dstv, m0, True)
                      _rmw_extreme(n0_v, dstv, m0, False)
                      _rmw_extreme(x1_v, dstv, m1, True)
                      _rmw_extreme(n1_v, dstv, m1, False)

              def vec_loop(i, cc):
                  ts = [prep(i * 8 * L + k * L) for k in range(8)]
                  for t in ts:
                      rmw(*t)
                  return cc
              lax.fori_loop(0, chunk // (8 * L), vec_loop, 0)
              return c
          lax.fori_loop(0, nchunks, chunk_loop, 0)

          pltpu.sync_copy(s0_v, sum_hbm.at[pl.ds(f_loc * nn, nn)])
          pltpu.sync_copy(s1_v, sum_hbm.at[pl.ds((f_loc + 1) * nn, nn)])
          pltpu.sync_copy(q0_v, sq_hbm.at[pl.ds(f_loc * nn, nn)])
          pltpu.sync_copy(q1_v, sq_hbm.at[pl.ds((f_loc + 1) * nn, nn)])
          pltpu.sync_copy(x0_v, mx_hbm.at[pl.ds(f_loc * nn, nn)])
          pltpu.sync_copy(x1_v, mx_hbm.at[pl.ds((f_loc + 1) * nn, nn)])
          pltpu.sync_copy(n0_v, mn_hbm.at[pl.ds(f_loc * nn, nn)])
          pltpu.sync_copy(n1_v, mn_hbm.at[pl.ds((f_loc + 1) * nn, nn)])

    return body


def _make_pair_body(n_nodes, n_pairs):
    ppw = n_pairs // NW

    def body(hp_hbm, pi_hbm, d2_hbm, hx_v, hy_v, hz_v, i0_v, i1_v, out_v):
        wid = lax.axis_index("c") * NS + lax.axis_index("s")
        base = wid * ppw
        nn = n_nodes
        pltpu.sync_copy(hp_hbm.at[pl.ds(0, nn)], hx_v)
        pltpu.sync_copy(hp_hbm.at[pl.ds(nn, nn)], hy_v)
        pltpu.sync_copy(hp_hbm.at[pl.ds(2 * nn, nn)], hz_v)
        pltpu.sync_copy(pi_hbm.at[pl.ds(base, ppw)], i0_v)
        pltpu.sync_copy(pi_hbm.at[pl.ds(n_pairs + base, ppw)], i1_v)

        def loop(i, c):
            sl = pl.ds(i * L, L)
            i0 = i0_v[sl]
            i1 = i1_v[sl]
            dx = plsc.load_gather(hx_v, [i0]) - plsc.load_gather(hx_v, [i1])
            dy = plsc.load_gather(hy_v, [i0]) - plsc.load_gather(hy_v, [i1])
            dz = plsc.load_gather(hz_v, [i0]) - plsc.load_gather(hz_v, [i1])
            out_v[sl] = dx * dx + dy * dy + dz * dz
            return c
        lax.fori_loop(0, ppw // L, loop, 0)
        pltpu.sync_copy(out_v, d2_hbm.at[pl.ds(base, ppw)])

    return body


_sc_mesh_cache = []


def _sc_mesh():
    if not _sc_mesh_cache:
        _sc_mesh_cache.append(plsc.VectorSubcoreMesh(
            core_axis_name="c", subcore_axis_name="s",
            num_cores=NC, num_subcores=NS,
        ))
    return _sc_mesh_cache[0]


def _full(shape):
    return pl.BlockSpec(shape, lambda i: (0, 0))


def kernel(x, pos, edge_index, edge_attr, pairwise_indices,
           W_in, b_in, W_msg, b_msg, W_upd, b_upd, W_proj, b_proj):
    n, d = x.shape
    e, ed = edge_attr.shape
    h_dim = W_in.shape[1]
    p = pairwise_indices.shape[1]
    del pos

    nb = 1024
    eb = 6400
    chunk = 640
    n_pad = ((n + nb - 1) // nb) * nb
    x = jnp.pad(x, ((0, n_pad - n), (0, 0)))

    f32 = jnp.float32
    w1 = W_msg[:h_dim]
    w2 = W_msg[h_dim:2 * h_dim]
    w3 = W_msg[2 * h_dim:]
    ea_t = edge_attr.T
    bin2 = b_in.reshape(1, h_dim)
    bmsg2 = b_msg.reshape(h_dim, 1)
    bupd2 = b_upd.reshape(1, h_dim)
    wp8 = jnp.pad(W_proj, ((0, 0), (0, 8 - W_proj.shape[1])))
    bp8 = jnp.pad(b_proj, (0, 8 - b_proj.shape[0])).reshape(8, 1)

    h, a_t, b_t = pl.pallas_call(
        _node_pre_body,
        grid=(n_pad // nb,),
        in_specs=[
            pl.BlockSpec((nb, d), lambda i: (i, 0)),
            _full((d, h_dim)), _full((1, h_dim)),
            _full((h_dim, h_dim)), _full((h_dim, h_dim)),
        ],
        out_specs=[
            pl.BlockSpec((nb, h_dim), lambda i: (i, 0)),
            pl.BlockSpec((h_dim, nb), lambda i: (0, i)),
            pl.BlockSpec((h_dim, nb), lambda i: (0, i)),
        ],
        out_shape=[
            jax.ShapeDtypeStruct((n_pad, h_dim), f32),
            jax.ShapeDtypeStruct((h_dim, n_pad), f32),
            jax.ShapeDtypeStruct((h_dim, n_pad), f32),
        ],
    )(x, W_in, bin2, w1, w2)

    c_t = pl.pallas_call(
        _edge_c_body,
        grid=(e // eb,),
        in_specs=[
            pl.BlockSpec((ed, eb), lambda i: (0, i)),
            _full((ed, h_dim)), _full((h_dim, 1)),
        ],
        out_specs=pl.BlockSpec((h_dim, eb), lambda i: (0, i)),
        out_shape=jax.ShapeDtypeStruct((h_dim, e), f32),
    )(ea_t, w3, bmsg2)

    src = edge_index[0]
    dst = edge_index[1]

    degp = pl.kernel(
        _make_deg_body(n_pad, e),
        compiler_params=pltpu.CompilerParams(needs_layout_passes=False),
        out_type=jax.ShapeDtypeStruct((NW * n_pad,), f32),
        mesh=_sc_mesh(),
        scratch_types=[
            pltpu.VMEM((e // NW,), jnp.int32),
            pltpu.VMEM((n_pad,), f32),
        ],
    )(dst)
    degp = degp.reshape(NW, n_pad)

    hw = h_dim // 2
    st, qt, xt, nt = pl.kernel(
        _make_edge_body(hw, n_pad, e, chunk),
        compiler_params=pltpu.CompilerParams(needs_layout_passes=False),
        out_type=[jax.ShapeDtypeStruct((h_dim * n_pad,), f32)] * 4,
        mesh=_sc_mesh(),
        scratch_types=(
            [pltpu.VMEM((n_pad,), f32)] * 4
            + [pltpu.VMEM((n_pad,), f32)] * 8
            + [pltpu.VMEM((2 * chunk,), jnp.int32)] * 2
            + [pltpu.VMEM((2 * chunk,), f32)] * 2
            + [pltpu.SemaphoreType.DMA] * 4
        ),
    )(a_t.reshape(-1), b_t.reshape(-1), c_t.reshape(-1), src, dst)
    st, qt, xt, nt = (r.reshape(h_dim, n_pad) for r in (st, qt, xt, nt))

    wu = [W_upd[i * h_dim:(i + 1) * h_dim] for i in range(4)]
    stat = pl.BlockSpec((h_dim, nb), lambda i: (0, i))
    hp = pl.pallas_call(
        _finalize_body,
        grid=(n_pad // nb,),
        in_specs=(
            [stat] * 4
            + [pl.BlockSpec((NW, nb), lambda i: (0, i)),
               pl.BlockSpec((nb, h_dim), lambda i: (i, 0))]
            + [_full((h_dim, h_dim))] * 4
            + [_full((1, h_dim)), _full((h_dim, 8)), _full((8, 1))]
        ),
        out_specs=pl.BlockSpec((8, nb), lambda i: (0, i)),
        out_shape=jax.ShapeDtypeStruct((8, n_pad), f32),
    )(st, qt, xt, nt, degp, h, *wu, bupd2, wp8, bp8)

    d2 = pl.kernel(
        _make_pair_body(n_pad, p),
        compiler_params=pltpu.CompilerParams(needs_layout_passes=False),
        out_type=jax.ShapeDtypeStruct((p,), f32),
        mesh=_sc_mesh(),
        scratch_types=(
            [pltpu.VMEM((n_pad,), f32)] * 3
            + [pltpu.VMEM((p // NW,), jnp.int32)] * 2
            + [pltpu.VMEM((p // NW,), f32)]
        ),
    )(hp.reshape(-1), pairwise_indices.reshape(-1))

    rows = 2500
    dist = pl.pallas_call(
        _sqrt_body,
        out_shape=jax.ShapeDtypeStruct((rows, p // rows), f32),
    )(d2.reshape(rows, p // rows))

    return dist.reshape(p, 1)
